# baseline (speedup 1.0000x reference)
import functools

import jax
import jax.numpy as jnp
from jax import lax
from jax.experimental import pallas as pl
from jax.experimental.pallas import tpu as pltpu
from jax.experimental.pallas import tpu_sc as plsc

_PROMPT_ROWS = 101
_NC = 2
_NS = 16
_NW = _NC * _NS


def _mask_body(mask_ref):
    mask_ref[...] = jnp.ones(mask_ref.shape, jnp.float32)


def _sc_tiled(emb, batch, embed_dim):
    b_per_w = batch // _NW
    mesh = plsc.VectorSubcoreMesh(core_axis_name="c", subcore_axis_name="s")

    @functools.partial(
        pl.kernel,
        mesh=mesh,
        out_type=jax.ShapeDtypeStruct((batch, _PROMPT_ROWS, embed_dim),
                                      jnp.float32),
        scratch_types=[
            pltpu.VMEM((_PROMPT_ROWS, embed_dim), jnp.float32),
            pltpu.SemaphoreType.DMA,
        ],
    )
    def k(emb_hbm, out_hbm, tab_v, sem):
        wid = lax.axis_index("s") * _NC + lax.axis_index("c")
        base = wid * b_per_w
        pltpu.sync_copy(emb_hbm, tab_v)

        def issue(i, _):
            pltpu.async_copy(tab_v, out_hbm.at[base + i], sem)
            return 0

        lax.fori_loop(0, b_per_w, issue, 0)

        def drain(i, _):
            pltpu.make_async_copy(tab_v, out_hbm.at[base + i], sem).wait()
            return 0

        lax.fori_loop(0, b_per_w, drain, 0)

    return k(emb)


def kernel(feature_map, key, embedding):
    del key
    batch = feature_map.shape[1]
    embed_dim = embedding.shape[1]
    emb = embedding[:_PROMPT_ROWS]

    tiled = _sc_tiled(emb, batch, embed_dim)
    mask = pl.pallas_call(
        _mask_body,
        out_shape=jax.ShapeDtypeStruct((batch, _PROMPT_ROWS), jnp.float32),
    )()
    return (tiled, mask)


# speedup vs baseline: 1.0089x; 1.0089x over previous
import functools

import jax
import jax.numpy as jnp
from jax import lax
from jax.experimental import pallas as pl
from jax.experimental.pallas import tpu as pltpu
from jax.experimental.pallas import tpu_sc as plsc

_PROMPT_ROWS = 101
_NC = 2
_NS = 16
_NW = _NC * _NS


def _mask_body(mask_ref):
    mask_ref[...] = jnp.ones(mask_ref.shape, jnp.float32)


def _sc_tiled(emb, batch, embed_dim):
    b_per_w = batch // _NW
    mesh = plsc.VectorSubcoreMesh(core_axis_name="c", subcore_axis_name="s")

    @functools.partial(
        pl.kernel,
        mesh=mesh,
        compiler_params=pltpu.CompilerParams(use_tc_tiling_on_sc=True),
        out_type=jax.ShapeDtypeStruct((batch, _PROMPT_ROWS, embed_dim),
                                      jnp.float32),
        scratch_types=[
            pltpu.VMEM((_PROMPT_ROWS, embed_dim), jnp.float32),
            pltpu.SemaphoreType.DMA,
        ],
    )
    def k(emb_hbm, out_hbm, tab_v, sem):
        wid = lax.axis_index("s") * _NC + lax.axis_index("c")
        base = wid * b_per_w
        pltpu.sync_copy(emb_hbm, tab_v)

        def issue(i, _):
            pltpu.async_copy(tab_v, out_hbm.at[base + i], sem)
            return 0

        lax.fori_loop(0, b_per_w, issue, 0)

        def drain(i, _):
            pltpu.make_async_copy(tab_v, out_hbm.at[base + i], sem).wait()
            return 0

        lax.fori_loop(0, b_per_w, drain, 0)

    return k(emb)


def kernel(feature_map, key, embedding):
    del key
    batch = feature_map.shape[1]
    embed_dim = embedding.shape[1]
    emb = embedding[:_PROMPT_ROWS]

    tiled = _sc_tiled(emb, batch, embed_dim)
    mask = pl.pallas_call(
        _mask_body,
        out_shape=jax.ShapeDtypeStruct((batch, _PROMPT_ROWS), jnp.float32),
    )()
    return (tiled, mask)
